# baseline (speedup 1.0000x reference)
import jax
import jax.numpy as jnp
from jax.experimental import pallas as pl
from jax.experimental.pallas import tpu as pltpu

_SPLITS = 2


def _mm_mask_kernel(emb_ref, att_ref, w_ref, out_ref):
    e = emb_ref[0]
    w = w_ref[...]
    logits_t = jax.lax.dot_general(
        w, e,
        dimension_numbers=(((1,), (1,)), ((), ())),
        preferred_element_type=jnp.float32,
    )
    att = att_ref[0]
    out_ref[0] = jnp.where(att, logits_t, -jnp.inf)


def kernel(emb_sentences, att_sentences, W):
    B, L, D = emb_sentences.shape
    NL = W.shape[0]
    LC = L // _SPLITS
    att3 = att_sentences.reshape(B, 1, L)

    out = pl.pallas_call(
        _mm_mask_kernel,
        grid=(B, _SPLITS),
        in_specs=[
            pl.BlockSpec((1, LC, D), lambda i, j: (i, j, 0)),
            pl.BlockSpec((1, 1, LC), lambda i, j: (i, 0, j)),
            pl.BlockSpec((NL, D), lambda i, j: (0, 0)),
        ],
        out_specs=pl.BlockSpec((1, NL, LC), lambda i, j: (i, 0, j)),
        out_shape=jax.ShapeDtypeStruct((B, NL, L), jnp.float32),
        compiler_params=pltpu.CompilerParams(
            dimension_semantics=("parallel", "parallel"),
        ),
    )(emb_sentences, att3, W)
    return out.transpose(0, 2, 1)


# speedup vs baseline: 1.1587x; 1.1587x over previous
import jax
import jax.numpy as jnp
from jax.experimental import pallas as pl
from jax.experimental.pallas import tpu as pltpu

_BB = 2


def _mm_mask_kernel(emb_ref, att_ref, w_ref, out_ref):
    w = w_ref[...]
    for h in range(_BB):
        e = emb_ref[h]
        logits_t = jax.lax.dot_general(
            w, e,
            dimension_numbers=(((1,), (1,)), ((), ())),
            preferred_element_type=jnp.float32,
        )
        att = att_ref[h]
        out_ref[h] = jnp.where(att, logits_t, -jnp.inf)


def kernel(emb_sentences, att_sentences, W):
    B, L, D = emb_sentences.shape
    NL = W.shape[0]
    att3 = att_sentences.reshape(B, 1, L)

    out = pl.pallas_call(
        _mm_mask_kernel,
        grid=(B // _BB,),
        in_specs=[
            pl.BlockSpec((_BB, L, D), lambda i: (i, 0, 0)),
            pl.BlockSpec((_BB, 1, L), lambda i: (i, 0, 0)),
            pl.BlockSpec((NL, D), lambda i: (0, 0)),
        ],
        out_specs=pl.BlockSpec((_BB, NL, L), lambda i: (i, 0, 0)),
        out_shape=jax.ShapeDtypeStruct((B, NL, L), jnp.float32),
        compiler_params=pltpu.CompilerParams(
            dimension_semantics=("parallel",),
        ),
    )(emb_sentences, att3, W)
    return out.transpose(0, 2, 1)


# speedup vs baseline: 1.1752x; 1.0142x over previous
import jax
import jax.numpy as jnp
from jax.experimental import pallas as pl
from jax.experimental.pallas import tpu as pltpu


def _mm_mask_kernel(emb_ref, att_ref, w_ref, out_ref):
    e = emb_ref[0]
    w = w_ref[...]
    logits_t = jax.lax.dot_general(
        w, e,
        dimension_numbers=(((1,), (1,)), ((), ())),
        preferred_element_type=jnp.float32,
    )
    att = att_ref[0]
    out_ref[0] = jnp.where(att, logits_t, -jnp.inf)


def kernel(emb_sentences, att_sentences, W):
    B, L, D = emb_sentences.shape
    NL = W.shape[0]
    att3 = att_sentences.reshape(B, 1, L)

    out = pl.pallas_call(
        _mm_mask_kernel,
        grid=(B,),
        in_specs=[
            pl.BlockSpec((1, L, D), lambda i: (i, 0, 0)),
            pl.BlockSpec((1, 1, L), lambda i: (i, 0, 0)),
            pl.BlockSpec((NL, D), lambda i: (0, 0)),
        ],
        out_specs=pl.BlockSpec((1, NL, L), lambda i: (i, 0, 0)),
        out_shape=jax.ShapeDtypeStruct((B, NL, L), jnp.float32),
        compiler_params=pltpu.CompilerParams(
            dimension_semantics=("parallel",),
        ),
    )(emb_sentences, att3, W)
    return out.transpose(0, 2, 1)
